# Initial kernel scaffold; baseline (speedup 1.0000x reference)
import jax
import jax.numpy as jnp
from jax.experimental import pallas as pl


def kernel(points, triangles, init_probs, W1a, b1a, W1b, b1b, W2a, b2a, W2b, b2b, W3a, b3a, W3b, b3b, Wf, bf):
    raise NotImplementedError("write your pallas kernel here")


# speedup vs baseline: 19.9483x; 19.9483x over previous
import functools

import jax
import jax.numpy as jnp
from jax import lax
from jax.experimental import pallas as pl
from jax.experimental.pallas import tpu as pltpu
from jax.experimental.pallas import tpu_sc as plsc

N = 10000
NP = 10240
KNN = 20
H = 128
NC, NS = 2, 16
NW = NC * NS
CH = NP // NW
NPTS = 5000
NPP = 5008
GRP = 4
NGRP = CH // GRP

_MESH = plsc.VectorSubcoreMesh(
    core_axis_name="c", subcore_axis_name="s", num_cores=NC, num_subcores=NS)


def _geom_body(px_h, py_h, pz_h, t0_h, t1_h, t2_h, g_h, b_h,
               pxv, pyv, pzv, t0v, t1v, t2v, gb, bb):
    w = lax.axis_index("s") * NC + lax.axis_index("c")
    base = w * CH
    pltpu.sync_copy(px_h, pxv)
    pltpu.sync_copy(py_h, pyv)
    pltpu.sync_copy(pz_h, pzv)
    pltpu.sync_copy(t0_h.at[pl.ds(base, CH)], t0v)
    pltpu.sync_copy(t1_h.at[pl.ds(base, CH)], t1v)
    pltpu.sync_copy(t2_h.at[pl.ds(base, CH)], t2v)
    for c in range(CH // 16):
        i0 = t0v[pl.ds(c * 16, 16)]
        i1 = t1v[pl.ds(c * 16, 16)]
        i2 = t2v[pl.ds(c * 16, 16)]
        p0x = plsc.load_gather(pxv, [i0])
        p1x = plsc.load_gather(pxv, [i1])
        p2x = plsc.load_gather(pxv, [i2])
        p0y = plsc.load_gather(pyv, [i0])
        p1y = plsc.load_gather(pyv, [i1])
        p2y = plsc.load_gather(pyv, [i2])
        p0z = plsc.load_gather(pzv, [i0])
        p1z = plsc.load_gather(pzv, [i1])
        p2z = plsc.load_gather(pzv, [i2])
        outs = []
        for (a, b, cc) in ((p0x, p1x, p2x), (p0y, p1y, p2y), (p0z, p1z, p2z)):
            e1 = a - b
            e2 = a - cc
            e3 = b - cc
            outs.append((jnp.minimum(jnp.minimum(e1, e2), e3),
                         jnp.maximum(jnp.maximum(e1, e2), e3),
                         ((a + b) + cc) / 3.0))
        vals = [outs[0][0], outs[1][0], outs[2][0],
                outs[0][1], outs[1][1], outs[2][1],
                outs[0][2], outs[1][2], outs[2][2]]
        lidx = lax.iota(jnp.int32, 16) + (c * 16)
        for d, v in enumerate(vals):
            plsc.store_scatter(gb, [lidx * 9 + d], v)
        bb[pl.ds(0 * CH + c * 16, 16)] = outs[0][2]
        bb[pl.ds(1 * CH + c * 16, 16)] = outs[1][2]
        bb[pl.ds(2 * CH + c * 16, 16)] = outs[2][2]
    pltpu.sync_copy(gb, g_h.at[pl.ds(base * 9, CH * 9)])
    for d in range(3):
        pltpu.sync_copy(bb.at[pl.ds(d * CH, CH)],
                        b_h.at[pl.ds(d * NP + base, CH)])


_SC_PARAMS = pltpu.CompilerParams(needs_layout_passes=False,
                                  use_tc_tiling_on_sc=False)

_geom = pl.kernel(
    _geom_body,
    out_type=[jax.ShapeDtypeStruct((NP * 9,), jnp.float32),
              jax.ShapeDtypeStruct((3 * NP,), jnp.float32)],
    mesh=_MESH,
    compiler_params=_SC_PARAMS,
    scratch_types=[pltpu.VMEM((NPP,), jnp.float32),
                   pltpu.VMEM((NPP,), jnp.float32),
                   pltpu.VMEM((NPP,), jnp.float32),
                   pltpu.VMEM((CH,), jnp.int32),
                   pltpu.VMEM((CH,), jnp.int32),
                   pltpu.VMEM((CH,), jnp.int32),
                   pltpu.VMEM((CH * 9,), jnp.float32),
                   pltpu.VMEM((3 * CH,), jnp.float32)],
)


def _knn_body(bt_ref, b2_ref, o_ref):
    bx = b2_ref[:, 0:1]
    by = b2_ref[:, 1:2]
    bz = b2_ref[:, 2:3]
    dx = bx - bt_ref[0:1, :]
    dy = by - bt_ref[1:2, :]
    dz = bz - bt_ref[2:3, :]
    d2 = dx * dx + dy * dy + dz * dz
    col = lax.broadcasted_iota(jnp.int32, (128, NP), 1)
    lane = lax.broadcasted_iota(jnp.int32, (128, 32), 1)
    acc = jnp.zeros((128, 32), jnp.int32)
    for t in range(KNN + 1):
        m = jnp.min(d2, axis=1, keepdims=True)
        idx = jnp.min(jnp.where(d2 == m, col, NP), axis=1, keepdims=True)
        acc = jnp.where(lane == t, idx, acc)
        d2 = jnp.where(col == idx, jnp.float32(jnp.inf), d2)
    o_ref[...] = acc


_knn = pl.pallas_call(
    _knn_body,
    grid=(NP // 128,),
    in_specs=[pl.BlockSpec((8, NP), lambda i: (0, 0)),
              pl.BlockSpec((128, 8), lambda i: (i, 0))],
    out_specs=pl.BlockSpec((128, 32), lambda i: (i, 0)),
    out_shape=jax.ShapeDtypeStruct((NP, 32), jnp.int32),
)


EG = 4
NEG = CH // EG
EW = CH * KNN
UD = 144


def _ebuild_body(u_h, tg_h, e_h, uown, tgv, gbuf, obuf, sem):
    w = lax.axis_index("s") * NC + lax.axis_index("c")
    base = w * CH
    pltpu.sync_copy(u_h.at[pl.ds(base, CH)], uown)
    pltpu.sync_copy(tg_h.at[pl.ds(base * KNN, EW)], tgv)

    def grp(gidx, carry):
        pltpu.async_copy(
            u_h.at[tgv.at[pl.ds(gidx * (EG * KNN), EG * KNN)]],
            gbuf, sem).wait()
        for n in range(EG):
            node = gidx * EG + n
            for r in range(UD // 16):
                own = uown[node, pl.ds(r * 16, 16)]
                for j in range(KNN):
                    e = n * KNN + j
                    obuf[e, pl.ds(r * 16, 16)] = own - gbuf[e, pl.ds(r * 16, 16)]
        pltpu.sync_copy(obuf, e_h.at[pl.ds(base * KNN + gidx * (EG * KNN), EG * KNN)])
        return carry

    lax.fori_loop(0, NEG, grp, 0)


_ebuild = pl.kernel(
    _ebuild_body,
    out_type=jax.ShapeDtypeStruct((NP * KNN, UD), jnp.float32),
    mesh=_MESH,
    compiler_params=_SC_PARAMS,
    scratch_types=[pltpu.VMEM((CH, UD), jnp.float32),
                   pltpu.VMEM((EW,), jnp.int32),
                   pltpu.VMEM((EG * KNN, UD), jnp.float32),
                   pltpu.VMEM((EG * KNN, UD), jnp.float32),
                   pltpu.SemaphoreType.DMA],
)


def _bfdot(a, b):
    return jnp.dot(a.astype(jnp.bfloat16), b.astype(jnp.bfloat16),
                   preferred_element_type=jnp.float32)


def _emlp_body(e_ref, wa_ref, ba_ref, wb_ref, bb_ref, o_ref):
    h = jnp.maximum(_bfdot(e_ref[...], wa_ref[...]) + ba_ref[...], 0.0)
    o_ref[...] = _bfdot(h, wb_ref[...]) + bb_ref[...]


_EB = 2048
NE = NP * KNN

_emlp = pl.pallas_call(
    _emlp_body, grid=(NE // _EB,),
    in_specs=[pl.BlockSpec((_EB, UD), lambda i: (i, 0)),
              pl.BlockSpec((UD, H), lambda i: (0, 0)),
              pl.BlockSpec((1, H), lambda i: (0, 0)),
              pl.BlockSpec((H, H), lambda i: (0, 0)),
              pl.BlockSpec((1, H), lambda i: (0, 0))],
    out_specs=pl.BlockSpec((_EB, H), lambda i: (i, 0)),
    out_shape=jax.ShapeDtypeStruct((NE, H), jnp.float32),
)


def _ssum_body(h_ref, o_ref):
    o_ref[...] = jnp.sum(h_ref[...], axis=1)


_SB = 256

_ssum = pl.pallas_call(
    _ssum_body, grid=(NP // _SB,),
    in_specs=[pl.BlockSpec((_SB, KNN, H), lambda i: (i, 0, 0))],
    out_specs=pl.BlockSpec((_SB, H), lambda i: (i, 0)),
    out_shape=jax.ShapeDtypeStruct((NP, H), jnp.float32),
)


def _head_body(x_ref, wf_ref, bf_ref, o_ref):
    o_ref[...] = jax.nn.sigmoid(_bfdot(x_ref[...], wf_ref[...]) + bf_ref[...])


_head = pl.pallas_call(
    _head_body, grid=(NP // 512,),
    in_specs=[pl.BlockSpec((512, H), lambda i: (i, 0)),
              pl.BlockSpec((H, H), lambda i: (0, 0)),
              pl.BlockSpec((1, H), lambda i: (0, 0))],
    out_specs=pl.BlockSpec((512, H), lambda i: (i, 0)),
    out_shape=jax.ShapeDtypeStruct((NP, H), jnp.float32),
)


def kernel(points, triangles, init_probs,
           W1a, b1a, W1b, b1b, W2a, b2a, W2b, b2b, W3a, b3a, W3b, b3b,
           Wf, bf):
    points = points.astype(jnp.float32)
    tri = triangles.astype(jnp.int32)
    px = jnp.pad(points[:, 0], (0, NPP - NPTS), constant_values=1e30)
    py = jnp.pad(points[:, 1], (0, NPP - NPTS), constant_values=1e30)
    pz = jnp.pad(points[:, 2], (0, NPP - NPTS), constant_values=1e30)
    tT = jnp.pad(tri.T, ((0, 0), (0, NP - N)), constant_values=NPTS)

    gflat, bflat = _geom(px, py, pz, tT[0], tT[1], tT[2])
    g9 = gflat.reshape(NP, 9)
    bt = bflat.reshape(3, NP)
    bt8 = jnp.pad(bt, ((0, 5), (0, 0)))
    b2 = jnp.pad(bt.T, ((0, 0), (0, 5)))

    nbr = _knn(bt8, b2)
    tg = jnp.pad(nbr[:N, 1:KNN + 1].reshape(-1), (0, (NP - N) * KNN))

    x = jnp.pad(jnp.broadcast_to(init_probs[:, None], (N, H)).astype(jnp.float32),
                ((0, NP - N), (0, 0)))
    zpad = jnp.zeros((NP, UD - 9 - H), jnp.float32)

    for (Wa, ba, Wb, bb) in ((W1a, b1a, W1b, b1b),
                             (W2a, b2a, W2b, b2b),
                             (W3a, b3a, W3b, b3b)):
        u = jnp.concatenate([g9, x, zpad], axis=1)
        E = _ebuild(u, tg)
        h2 = _emlp(E, jnp.pad(Wa, ((0, UD - 9 - H), (0, 0))),
                   ba.reshape(1, H), Wb, bb.reshape(1, H))
        x = _ssum(h2.reshape(NP, KNN, H))
        x = jnp.pad(x[:N], ((0, NP - N), (0, 0)))

    wfp = jnp.pad(Wf, ((0, 0), (0, H - 1)))
    bfp = jnp.broadcast_to(bf[None, :], (1, H))
    out = _head(x, wfp, bfp)
    return out[:N, 0]
